# baseline (speedup 1.0000x reference)
import functools

import jax
import jax.numpy as jnp
import numpy as np
from jax import lax
from jax.experimental import pallas as pl
from jax.experimental.pallas import tpu as pltpu
from jax.experimental.pallas import tpu_sc as plsc

NUM_LEVELS = 16
LEVEL_DIM = 2
HSZ = 1 << 19
P1 = 2654435761 - (1 << 32)
P2 = 805459861
B_TOTAL = 262144
OUT_DIM = NUM_LEVELS * LEVEL_DIM


def _offsets():
    offs, off = [], 0
    for i in range(NUM_LEVELS):
        res = 16 << i
        offs.append(off)
        off += min(HSZ, res ** 3)
    offs.append(off)
    return offs


_OFF = _offsets()
_N_EMBED = _OFF[-1]

NC, NS = 2, 16
NW = NC * NS
LANES = 16


def _make_sc_call(batch, chunk, interpret=False):
    pts = batch // NW
    nch = pts // chunk
    nv = chunk // LANES
    mesh = plsc.VectorSubcoreMesh(core_axis_name="c", subcore_axis_name="s")

    @functools.partial(
        pl.kernel,
        out_type=jax.ShapeDtypeStruct((OUT_DIM, batch), jnp.float32),
        mesh=mesh,
        interpret=interpret,
        scratch_types=[
            pltpu.VMEM((chunk,), jnp.float32),
            pltpu.VMEM((chunk,), jnp.float32),
            pltpu.VMEM((chunk,), jnp.float32),
            pltpu.VMEM((16 * chunk,), jnp.int32),
            pltpu.VMEM((8 * chunk,), jnp.float32),
            pltpu.VMEM((16 * chunk,), jnp.float32),
            pltpu.VMEM((2, chunk), jnp.float32),
            pltpu.SemaphoreType.DMA,
        ],
    )
    def sc_encode(x_hbm, y_hbm, z_hbm, tab_hbm, out_hbm,
                  xv, yv, zv, idxv, wv, rows, outv, sem):
        wid = lax.axis_index("s") * NC + lax.axis_index("c")
        base = wid * pts

        def do_level(scale_f, off, mask, use_hash, r_lin, col0, pt0):
            def phase_a(v, carry):
                i0 = v * LANES
                xi = xv[pl.ds(i0, LANES)]
                yi = yv[pl.ds(i0, LANES)]
                zi = zv[pl.ds(i0, LANES)]
                px = xi * scale_f + 0.5
                py = yi * scale_f + 0.5
                pz = zi * scale_f + 0.5
                gx = px.astype(jnp.int32)
                gy = py.astype(jnp.int32)
                gz = pz.astype(jnp.int32)
                fx = px - gx.astype(jnp.float32)
                fy = py - gy.astype(jnp.float32)
                fz = pz - gz.astype(jnp.float32)
                if use_hash:
                    tx0, tx1 = gx, gx + 1
                    ty0 = gy * P1
                    ty1 = ty0 + P1
                    tz0 = gz * P2
                    tz1 = tz0 + P2
                    comb = lambda a, b, c: a ^ b ^ c
                else:
                    sy, sz = r_lin, r_lin * r_lin
                    tx0, tx1 = gx, gx + 1
                    ty0 = gy * sy
                    ty1 = ty0 + sy
                    tz0 = gz * sz
                    tz1 = tz0 + sz
                    comb = lambda a, b, c: a + b + c
                txs, tys, tzs = (tx0, tx1), (ty0, ty1), (tz0, tz1)
                wxs = (1.0 - fx, fx)
                wys = (1.0 - fy, fy)
                wzs = (1.0 - fz, fz)
                for c in range(8):
                    bx, by, bz = c & 1, (c >> 1) & 1, (c >> 2) & 1
                    idx = (comb(txs[bx], tys[by], tzs[bz]) & mask) + off
                    w = wxs[bx] * wys[by] * wzs[bz]
                    wi = idx * 2
                    idxv[pl.ds(2 * c * chunk + i0, LANES)] = wi
                    idxv[pl.ds(2 * c * chunk + chunk + i0, LANES)] = wi + 1
                    wv[pl.ds(c * chunk + i0, LANES)] = w
                return carry

            lax.fori_loop(0, nv, phase_a, 0, unroll=False)
            pltpu.async_copy(tab_hbm.at[idxv], rows, sem).wait()

            def phase_b(v, carry):
                i0 = v * LANES
                acc0 = jnp.zeros((LANES,), jnp.float32)
                acc1 = jnp.zeros((LANES,), jnp.float32)
                for c in range(8):
                    w = wv[pl.ds(c * chunk + i0, LANES)]
                    e0 = rows[pl.ds(2 * c * chunk + i0, LANES)]
                    e1 = rows[pl.ds(2 * c * chunk + chunk + i0, LANES)]
                    acc0 = acc0 + w * e0
                    acc1 = acc1 + w * e1
                outv[0, pl.ds(i0, LANES)] = acc0
                outv[1, pl.ds(i0, LANES)] = acc1
                return carry

            lax.fori_loop(0, nv, phase_b, 0, unroll=False)
            pltpu.sync_copy(
                outv, out_hbm.at[pl.ds(col0, 2), pl.ds(pt0, chunk)]
            )

        def chunk_body(ch, carry):
            pt0 = base + ch * chunk
            pltpu.sync_copy(x_hbm.at[pl.ds(pt0, chunk)], xv)
            pltpu.sync_copy(y_hbm.at[pl.ds(pt0, chunk)], yv)
            pltpu.sync_copy(z_hbm.at[pl.ds(pt0, chunk)], zv)
            for l in range(3):
                r = 16 << l
                do_level(np.float32(r - 1), _OFF[l], r ** 3 - 1, False, r,
                         2 * l, pt0)

            def hash_level(l, c2):
                scale_f = (jnp.left_shift(16, l) - 1).astype(jnp.float32)
                off = _OFF[3] + (l - 3) * HSZ
                do_level(scale_f, off, HSZ - 1, True, 0, 2 * l, pt0)
                return c2

            lax.fori_loop(3, NUM_LEVELS, hash_level, 0, unroll=False)
            return carry

        lax.fori_loop(0, nch, chunk_body, 0, unroll=False)

    return sc_encode


_sc_call = _make_sc_call(B_TOTAL, 512)


@jax.jit
def kernel(inputs, embeddings):
    xt = inputs.T
    out_t = _sc_call(xt[0], xt[1], xt[2], embeddings.reshape(-1))
    return out_t.T
